# Initial kernel scaffold; baseline (speedup 1.0000x reference)
import functools

import jax
import jax.numpy as jnp
from jax import lax
from jax.experimental import pallas as pl
from jax.experimental.pallas import tpu as pltpu
from jax.experimental.pallas import tpu_sc as plsc

VOCAB = 32000
D_MODEL = 2048
SEQ = 2048
EPS = 1e-5

TV = 640
NV = VOCAB // TV
NC = 2
NS = 16
NW = NC * NS
BPW = SEQ // NW
CH = 32


def _gather_body(x_hbm, table_hbm, out_hbm, idx_v, rows_v, sem):
    wid = lax.axis_index("s") * NC + lax.axis_index("c")
    base = wid * BPW
    pltpu.sync_copy(x_hbm.at[pl.ds(base, BPW)], idx_v)
    for c in range(BPW // CH):
        pltpu.async_copy(
            table_hbm.at[idx_v.at[pl.ds(c * CH, CH)]], rows_v, sem
        ).wait()
        pltpu.sync_copy(rows_v, out_hbm.at[pl.ds(base + c * CH, CH)])


def _ln_matmul_body(h_ref, g_ref, b_ref, emb_ref, out_ref, hn_ref):
    @pl.when(pl.program_id(0) == 0)
    def _():
        h = h_ref[...]
        mu = jnp.mean(h, axis=1, keepdims=True)
        cen = h - mu
        var = jnp.mean(cen * cen, axis=1, keepdims=True)
        hn = cen * lax.rsqrt(var + EPS) * g_ref[...] + b_ref[...]
        hn_ref[...] = hn.astype(jnp.bfloat16)

    out_ref[...] = lax.dot_general(
        hn_ref[...],
        emb_ref[...].astype(jnp.bfloat16),
        (((1,), (1,)), ((), ())),
        preferred_element_type=jnp.float32,
    )


def kernel(x, embedding, ln_gamma, ln_beta):
    xi = x.reshape(SEQ).astype(jnp.int32)

    gather = functools.partial(
        pl.kernel,
        mesh=plsc.VectorSubcoreMesh(core_axis_name="c", subcore_axis_name="s"),
        out_type=jax.ShapeDtypeStruct((SEQ, D_MODEL), jnp.float32),
        scratch_types=[
            pltpu.VMEM((BPW,), jnp.int32),
            pltpu.VMEM((CH, D_MODEL), jnp.float32),
            pltpu.SemaphoreType.DMA,
        ],
    )(_gather_body)
    h = gather(xi, embedding)

    logits = pl.pallas_call(
        _ln_matmul_body,
        grid=(NV,),
        in_specs=[
            pl.BlockSpec((SEQ, D_MODEL), lambda v: (0, 0)),
            pl.BlockSpec((1, D_MODEL), lambda v: (0, 0)),
            pl.BlockSpec((1, D_MODEL), lambda v: (0, 0)),
            pl.BlockSpec((TV, D_MODEL), lambda v: (v, 0)),
        ],
        out_specs=pl.BlockSpec((SEQ, TV), lambda v: (0, v)),
        out_shape=jax.ShapeDtypeStruct((SEQ, VOCAB), jnp.float32),
        scratch_shapes=[pltpu.VMEM((SEQ, D_MODEL), jnp.bfloat16)],
        compiler_params=pltpu.CompilerParams(
            dimension_semantics=("arbitrary",)
        ),
    )(h, ln_gamma.reshape(1, D_MODEL), ln_beta.reshape(1, D_MODEL), embedding)

    return (logits.reshape(1, SEQ, VOCAB), 0.0)


# speedup vs baseline: 1.0214x; 1.0214x over previous
import functools

import jax
import jax.numpy as jnp
from jax import lax
from jax.experimental import pallas as pl
from jax.experimental.pallas import tpu as pltpu
from jax.experimental.pallas import tpu_sc as plsc

VOCAB = 32000
D_MODEL = 2048
SEQ = 2048
EPS = 1e-5

TV = 640
TS = 256
NV = VOCAB // TV
NC = 2
NS = 16
NW = NC * NS
BPW = SEQ // NW
CH = 32


def _gather_body(x_hbm, table_hbm, out_hbm, idx_v, rows_v, sem):
    wid = lax.axis_index("s") * NC + lax.axis_index("c")
    base = wid * BPW
    pltpu.sync_copy(x_hbm.at[pl.ds(base, BPW)], idx_v)
    for c in range(BPW // CH):
        pltpu.async_copy(
            table_hbm.at[idx_v.at[pl.ds(c * CH, CH)]], rows_v, sem
        ).wait()
        pltpu.sync_copy(rows_v, out_hbm.at[pl.ds(base + c * CH, CH)])


def _ln_body(h_ref, g_ref, b_ref, hn_ref):
    h = h_ref[...]
    mu = jnp.mean(h, axis=1, keepdims=True)
    cen = h - mu
    var = jnp.mean(cen * cen, axis=1, keepdims=True)
    hn = cen * lax.rsqrt(var + EPS) * g_ref[...] + b_ref[...]
    hn_ref[...] = hn.astype(jnp.bfloat16)


def _matmul_body(hn_ref, emb_ref, out_ref):
    out_ref[...] = lax.dot_general(
        hn_ref[...],
        emb_ref[...].astype(jnp.bfloat16),
        (((1,), (1,)), ((), ())),
        preferred_element_type=jnp.float32,
    )


def kernel(x, embedding, ln_gamma, ln_beta):
    xi = x.reshape(SEQ).astype(jnp.int32)

    gather = functools.partial(
        pl.kernel,
        mesh=plsc.VectorSubcoreMesh(core_axis_name="c", subcore_axis_name="s"),
        out_type=jax.ShapeDtypeStruct((SEQ, D_MODEL), jnp.float32),
        scratch_types=[
            pltpu.VMEM((BPW,), jnp.int32),
            pltpu.VMEM((CH, D_MODEL), jnp.float32),
            pltpu.SemaphoreType.DMA,
        ],
    )(_gather_body)
    h = gather(xi, embedding)

    hn = pl.pallas_call(
        _ln_body,
        grid=(SEQ // TS,),
        in_specs=[
            pl.BlockSpec((TS, D_MODEL), lambda s: (s, 0)),
            pl.BlockSpec((1, D_MODEL), lambda s: (0, 0)),
            pl.BlockSpec((1, D_MODEL), lambda s: (0, 0)),
        ],
        out_specs=pl.BlockSpec((TS, D_MODEL), lambda s: (s, 0)),
        out_shape=jax.ShapeDtypeStruct((SEQ, D_MODEL), jnp.bfloat16),
        compiler_params=pltpu.CompilerParams(
            dimension_semantics=("parallel",)
        ),
    )(h, ln_gamma.reshape(1, D_MODEL), ln_beta.reshape(1, D_MODEL))

    logits = pl.pallas_call(
        _matmul_body,
        grid=(NV,),
        in_specs=[
            pl.BlockSpec((SEQ, D_MODEL), lambda v: (0, 0)),
            pl.BlockSpec((TV, D_MODEL), lambda v: (v, 0)),
        ],
        out_specs=pl.BlockSpec((SEQ, TV), lambda v: (0, v)),
        out_shape=jax.ShapeDtypeStruct((SEQ, VOCAB), jnp.float32),
        compiler_params=pltpu.CompilerParams(
            dimension_semantics=("parallel",)
        ),
    )(hn, embedding)

    return (logits.reshape(1, SEQ, VOCAB), 0.0)


# speedup vs baseline: 1.1618x; 1.1375x over previous
import functools

import jax
import jax.numpy as jnp
from jax import lax
from jax.experimental import pallas as pl
from jax.experimental.pallas import tpu as pltpu
from jax.experimental.pallas import tpu_sc as plsc

VOCAB = 32000
D_MODEL = 2048
SEQ = 2048
EPS = 1e-5

TV = 512
TS = 256
NV = -(-VOCAB // TV)
NC = 2
NS = 16
NW = NC * NS
BPW = SEQ // NW
CH = 32


def _gather_body(x_hbm, table_hbm, out_hbm, idx_v, rows_v, sem):
    wid = lax.axis_index("s") * NC + lax.axis_index("c")
    base = wid * BPW
    pltpu.sync_copy(x_hbm.at[pl.ds(base, BPW)], idx_v)
    for c in range(BPW // CH):
        pltpu.async_copy(
            table_hbm.at[idx_v.at[pl.ds(c * CH, CH)]], rows_v, sem
        ).wait()
        pltpu.sync_copy(rows_v, out_hbm.at[pl.ds(base + c * CH, CH)])


def _ln_body(h_ref, g_ref, b_ref, hn_ref):
    h = h_ref[...]
    mu = jnp.mean(h, axis=1, keepdims=True)
    cen = h - mu
    var = jnp.mean(cen * cen, axis=1, keepdims=True)
    hn = cen * lax.rsqrt(var + EPS) * g_ref[...] + b_ref[...]
    hn_ref[...] = hn.astype(jnp.bfloat16)


def _matmul_body(hn_ref, emb_ref, out_ref):
    out_ref[...] = lax.dot_general(
        hn_ref[...],
        emb_ref[...].astype(jnp.bfloat16),
        (((1,), (1,)), ((), ())),
        preferred_element_type=jnp.float32,
    )


def kernel(x, embedding, ln_gamma, ln_beta):
    xi = x.reshape(SEQ).astype(jnp.int32)

    gather = functools.partial(
        pl.kernel,
        mesh=plsc.VectorSubcoreMesh(core_axis_name="c", subcore_axis_name="s"),
        out_type=jax.ShapeDtypeStruct((SEQ, D_MODEL), jnp.float32),
        scratch_types=[
            pltpu.VMEM((BPW,), jnp.int32),
            pltpu.VMEM((CH, D_MODEL), jnp.float32),
            pltpu.SemaphoreType.DMA,
        ],
    )(_gather_body)
    h = gather(xi, embedding)

    hn = pl.pallas_call(
        _ln_body,
        grid=(SEQ // TS,),
        in_specs=[
            pl.BlockSpec((TS, D_MODEL), lambda s: (s, 0)),
            pl.BlockSpec((1, D_MODEL), lambda s: (0, 0)),
            pl.BlockSpec((1, D_MODEL), lambda s: (0, 0)),
        ],
        out_specs=pl.BlockSpec((TS, D_MODEL), lambda s: (s, 0)),
        out_shape=jax.ShapeDtypeStruct((SEQ, D_MODEL), jnp.bfloat16),
        compiler_params=pltpu.CompilerParams(
            dimension_semantics=("parallel",)
        ),
    )(h, ln_gamma.reshape(1, D_MODEL), ln_beta.reshape(1, D_MODEL))

    logits = pl.pallas_call(
        _matmul_body,
        grid=(NV,),
        in_specs=[
            pl.BlockSpec((SEQ, D_MODEL), lambda v: (0, 0)),
            pl.BlockSpec((TV, D_MODEL), lambda v: (v, 0)),
        ],
        out_specs=pl.BlockSpec((SEQ, TV), lambda v: (0, v)),
        out_shape=jax.ShapeDtypeStruct((SEQ, VOCAB), jnp.float32),
        compiler_params=pltpu.CompilerParams(
            dimension_semantics=("parallel",)
        ),
    )(hn, embedding)

    return (logits.reshape(1, SEQ, VOCAB), 0.0)
